# baseline (speedup 1.0000x reference)
import functools

import jax
import jax.numpy as jnp
from jax import lax
from jax.experimental import pallas as pl
from jax.experimental.pallas import tpu as pltpu
from jax.experimental.pallas import tpu_sc as plsc

B = 16384
D = 128
V = 8
NC = 2
NS = 16
NW = NC * NS
B_PER_W = B // NW
CHUNK = 64
N_CHUNKS = B_PER_W // CHUNK
REP_PER_TILE = 4
N_REP = NS * REP_PER_TILE
GRP = REP_PER_TILE * V
SPLIT = 5


def _body(names_hbm, emb_hbm, out_hbm, table_sh, idx_v, rows_v,
          sem_i, sem_s, sem_g, sem_w):
    sid = lax.axis_index("s")
    wid = sid * NC + lax.axis_index("c")
    base = wid * B_PER_W

    pltpu.async_copy(names_hbm.at[pl.ds(base, B_PER_W)], idx_v, sem_i)
    pltpu.async_copy(
        emb_hbm.at[pl.ds(wid * GRP, GRP)],
        table_sh.at[pl.ds(sid * GRP, GRP)],
        sem_s,
    )
    pltpu.make_async_copy(
        emb_hbm.at[pl.ds(wid * GRP, GRP)],
        table_sh.at[pl.ds(sid * GRP, GRP)],
        sem_s,
    ).wait()
    pltpu.make_async_copy(names_hbm.at[pl.ds(base, B_PER_W)], idx_v, sem_i).wait()
    plsc.subcore_barrier()
    for j in range(N_CHUNKS):
        src = table_sh if j < SPLIT else emb_hbm
        pltpu.async_copy(
            src.at[idx_v.at[pl.ds(j * CHUNK, CHUNK)]],
            rows_v.at[pl.ds(j * CHUNK, CHUNK)],
            sem_g.at[j],
        )
    for j in range(N_CHUNKS):
        src = table_sh if j < SPLIT else emb_hbm
        pltpu.make_async_copy(
            src.at[idx_v.at[pl.ds(j * CHUNK, CHUNK)]],
            rows_v.at[pl.ds(j * CHUNK, CHUNK)],
            sem_g.at[j],
        ).wait()
        pltpu.async_copy(
            rows_v.at[pl.ds(j * CHUNK, CHUNK)],
            out_hbm.at[pl.ds(base + j * CHUNK, CHUNK)],
            sem_w,
        )
    for j in range(N_CHUNKS):
        pltpu.make_async_copy(
            rows_v.at[pl.ds(j * CHUNK, CHUNK)],
            out_hbm.at[pl.ds(base + j * CHUNK, CHUNK)],
            sem_w,
        ).wait()


@jax.jit
def kernel(names, emb):
    mesh = plsc.VectorSubcoreMesh(core_axis_name="c", subcore_axis_name="s")
    f = pl.kernel(
        _body,
        out_type=jax.ShapeDtypeStruct((B, D), jnp.float32),
        mesh=mesh,
        scratch_types=[
            pltpu.VMEM_SHARED((N_REP * V, D), jnp.float32),
            pltpu.VMEM((B_PER_W,), jnp.int32),
            pltpu.VMEM((B_PER_W, D), jnp.float32),
            pltpu.SemaphoreType.DMA,
            pltpu.SemaphoreType.DMA,
            pltpu.SemaphoreType.DMA((N_CHUNKS,)),
            pltpu.SemaphoreType.DMA,
        ],
    )
    emb_rep = jnp.tile(emb, (NW * REP_PER_TILE, 1))
    p = lax.iota(jnp.int32, B)
    wid = p // B_PER_W
    sid = wid // NC
    chunk_in_w = (p % B_PER_W) // CHUNK
    spmem_off = (sid * REP_PER_TILE + (p % REP_PER_TILE)) * V
    hbm_off = (wid * REP_PER_TILE + (p % REP_PER_TILE)) * V
    rep_off = jnp.where(chunk_in_w < SPLIT, spmem_off, hbm_off)
    return f(names.astype(jnp.int32) + rep_off, emb_rep)


# speedup vs baseline: 1.0799x; 1.0799x over previous
import functools

import jax
import jax.numpy as jnp
from jax import lax
from jax.experimental import pallas as pl
from jax.experimental.pallas import tpu as pltpu
from jax.experimental.pallas import tpu_sc as plsc

B = 16384
D = 128
V = 8
NC = 2
NS = 16
NW = NC * NS
B_PER_W = B // NW
CHUNK = 64
N_CHUNKS = B_PER_W // CHUNK
REP_PER_TILE = 4
N_REP = NS * REP_PER_TILE
GRP = REP_PER_TILE * V


def _body(names_hbm, emb_hbm, out_hbm, table_sh, idx_v, rows_v,
          sem_i, sem_s, sem_g, sem_w):
    sid = lax.axis_index("s")
    wid = sid * NC + lax.axis_index("c")
    base = wid * B_PER_W

    pltpu.async_copy(names_hbm.at[pl.ds(base, B_PER_W)], idx_v, sem_i)
    pltpu.async_copy(
        emb_hbm.at[pl.ds(wid * GRP, GRP)],
        table_sh.at[pl.ds(sid * GRP, GRP)],
        sem_s,
    )
    pltpu.make_async_copy(
        emb_hbm.at[pl.ds(wid * GRP, GRP)],
        table_sh.at[pl.ds(sid * GRP, GRP)],
        sem_s,
    ).wait()
    pltpu.make_async_copy(names_hbm.at[pl.ds(base, B_PER_W)], idx_v, sem_i).wait()
    plsc.subcore_barrier()
    for j in range(N_CHUNKS):
        pltpu.async_copy(
            table_sh.at[idx_v.at[pl.ds(j * CHUNK, CHUNK)]],
            rows_v.at[pl.ds(j * CHUNK, CHUNK)],
            sem_g.at[j],
        )
    for j in range(N_CHUNKS):
        pltpu.make_async_copy(
            table_sh.at[idx_v.at[pl.ds(j * CHUNK, CHUNK)]],
            rows_v.at[pl.ds(j * CHUNK, CHUNK)],
            sem_g.at[j],
        ).wait()
        pltpu.async_copy(
            rows_v.at[pl.ds(j * CHUNK, CHUNK)],
            out_hbm.at[pl.ds(base + j * CHUNK, CHUNK)],
            sem_w,
        )
    for j in range(N_CHUNKS):
        pltpu.make_async_copy(
            rows_v.at[pl.ds(j * CHUNK, CHUNK)],
            out_hbm.at[pl.ds(base + j * CHUNK, CHUNK)],
            sem_w,
        ).wait()


@jax.jit
def kernel(names, emb):
    mesh = plsc.VectorSubcoreMesh(core_axis_name="c", subcore_axis_name="s")
    f = pl.kernel(
        _body,
        out_type=jax.ShapeDtypeStruct((B, D), jnp.float32),
        mesh=mesh,
        scratch_types=[
            pltpu.VMEM_SHARED((N_REP * V, D), jnp.float32),
            pltpu.VMEM((B_PER_W,), jnp.int32),
            pltpu.VMEM((B_PER_W, D), jnp.float32),
            pltpu.SemaphoreType.DMA,
            pltpu.SemaphoreType.DMA,
            pltpu.SemaphoreType.DMA((N_CHUNKS,)),
            pltpu.SemaphoreType.DMA,
        ],
    )
    emb_rep = jnp.tile(emb, (NW * REP_PER_TILE, 1))
    p = lax.iota(jnp.int32, B)
    sid = (p // B_PER_W) // NC
    rep_off = (sid * REP_PER_TILE + (p % REP_PER_TILE)) * V
    return f(names.astype(jnp.int32) + rep_off, emb_rep)
